# baseline (speedup 1.0000x reference)
import functools

import jax
import jax.numpy as jnp
from jax import lax
from jax.experimental import pallas as pl
from jax.experimental.pallas import tpu as pltpu
from jax.experimental.pallas import tpu_sc as plsc

N = 10000
D = 128
DL = 16
E = 320000
NC = 2
NS = 16
NW = NC * NS
EPW = E // NW
B = 125
NB = EPW // B
RPT = N // NS


def _make_agg_body(with_deg):
    def body(tab_hbm, src_hbm, dst_hbm, zz_hbm, *rest):
        if with_deg:
            (zd_hbm, ones_hbm, agg_hbm, deg_hbm,
             sb0, db0, sb1, db1, sb2, db2, sb3, db3,
             rows0, rows1, onesb, acc, degacc,
             semi0, semi1, semi2, semi3, semg0, semg0b, semg1, semg1b,
             sems0, sems1, semz) = rest
        else:
            (agg_hbm,
             sb0, db0, sb1, db1, sb2, db2, sb3, db3,
             rows0, rows1, acc,
             semi0, semi1, semi2, semi3, semg0, semg0b, semg1, semg1b,
             sems0, sems1, semz) = rest
        c = lax.axis_index("c")
        s = lax.axis_index("s")
        wid = s * NC + c
        pltpu.async_copy(zz_hbm, acc.at[pl.ds(s * RPT, RPT)], semz)
        if with_deg:
            pltpu.async_copy(zd_hbm, degacc.at[pl.ds(s * RPT, RPT)], semz)
            pltpu.async_copy(ones_hbm, onesb, semz)

        ibase = wid * NB

        def iload(i, sb, db, sem):
            pltpu.async_copy(src_hbm.at[ibase + i], sb, sem)
            pltpu.async_copy(dst_hbm.at[ibase + i], db, sem)

        def iwait(sb, db, sem):
            pltpu.make_async_copy(src_hbm.at[0], sb, sem).wait()
            pltpu.make_async_copy(dst_hbm.at[0], db, sem).wait()

        GH = 64

        def gather(sb, rows, sem, semb):
            pltpu.async_copy(tab_hbm.at[sb.at[pl.ds(0, GH)]],
                             rows.at[pl.ds(0, GH)], sem)
            pltpu.async_copy(tab_hbm.at[sb.at[pl.ds(GH, B - GH)]],
                             rows.at[pl.ds(GH, B - GH)], semb)

        def gwait(rows, sem, semb):
            pltpu.make_async_copy(tab_hbm.at[sb0.at[pl.ds(0, GH)]],
                                  rows.at[pl.ds(0, GH)], sem).wait()
            pltpu.make_async_copy(tab_hbm.at[sb0.at[pl.ds(GH, B - GH)]],
                                  rows.at[pl.ds(GH, B - GH)], semb).wait()

        def scat(db, rows, sem):
            pltpu.async_copy(rows, acc.at[db], sem, add=True)
            if with_deg:
                pltpu.async_copy(onesb, degacc.at[db], sem, add=True)

        def swait(db, rows, sem):
            pltpu.make_async_copy(rows, acc.at[db], sem).wait()
            if with_deg:
                pltpu.make_async_copy(onesb, degacc.at[db], sem).wait()

        iload(0, sb0, db0, semi0)
        iload(1, sb1, db1, semi1)
        iload(2, sb2, db2, semi2)
        iload(3, sb3, db3, semi3)
        iwait(sb0, db0, semi0)
        gather(sb0, rows0, semg0, semg0b)
        iwait(sb1, db1, semi1)
        gather(sb1, rows1, semg1, semg1b)
        pltpu.make_async_copy(zz_hbm, acc.at[pl.ds(s * RPT, RPT)],
                              semz).wait()
        if with_deg:
            pltpu.make_async_copy(zd_hbm, degacc.at[pl.ds(s * RPT, RPT)],
                                  semz).wait()
            pltpu.make_async_copy(ones_hbm, onesb, semz).wait()
        plsc.subcore_barrier()

        def quarter(i_next, sb_c, db_c, semi_c, sb_n, db_n, semi_n,
                    rows, semg, semgb, sems):
            gwait(rows, semg, semgb)
            scat(db_c, rows, sems)
            swait(db_c, rows, sems)
            iload(i_next, sb_c, db_c, semi_c)
            iwait(sb_n, db_n, semi_n)
            gather(sb_n, rows, semg, semgb)

        def step(j, carry):
            i0 = 4 * j
            quarter(i0 + 4, sb0, db0, semi0, sb2, db2, semi2,
                    rows0, semg0, semg0b, sems0)
            quarter(i0 + 5, sb1, db1, semi1, sb3, db3, semi3,
                    rows1, semg1, semg1b, sems1)
            quarter(i0 + 6, sb2, db2, semi2, sb0, db0, semi0,
                    rows0, semg0, semg0b, sems0)
            quarter(i0 + 7, sb3, db3, semi3, sb1, db1, semi1,
                    rows1, semg1, semg1b, sems1)
            return carry

        lax.fori_loop(0, NB // 4 - 1, step, 0)
        gwait(rows0, semg0, semg0b)
        scat(db0, rows0, sems0)
        swait(db0, rows0, sems0)
        iwait(sb2, db2, semi2)
        gather(sb2, rows0, semg0, semg0b)
        gwait(rows1, semg1, semg1b)
        scat(db1, rows1, sems1)
        swait(db1, rows1, sems1)
        iwait(sb3, db3, semi3)
        gather(sb3, rows1, semg1, semg1b)
        gwait(rows0, semg0, semg0b)
        scat(db2, rows0, sems0)
        swait(db2, rows0, sems0)
        gwait(rows1, semg1, semg1b)
        scat(db3, rows1, sems1)
        swait(db3, rows1, sems1)
        plsc.subcore_barrier()
        pltpu.sync_copy(acc.at[pl.ds(s * RPT, RPT)], agg_hbm.at[c * NS + s])
        if with_deg:
            pltpu.sync_copy(degacc.at[pl.ds(s * RPT, RPT)],
                            deg_hbm.at[c * NS + s])

    return body


_SC_MESH = dict(
    mesh=plsc.VectorSubcoreMesh(core_axis_name="c", subcore_axis_name="s"),
    compiler_params=pltpu.CompilerParams(use_tc_tiling_on_sc=False),
)

_IDX_SCRATCH = (
    (pltpu.VMEM((B,), jnp.int32),) * 8
    + (
        pltpu.VMEM((B, D), jnp.float32),
        pltpu.VMEM((B, D), jnp.float32),
    )
)

_SEMS = (pltpu.SemaphoreType.DMA,) * 11


def _sc_agg_deg(table, src, dst, zz, zd, ones_in):
    return pl.kernel(
        _make_agg_body(True),
        out_type=(jax.ShapeDtypeStruct((NW, RPT, D), jnp.float32),
                  jax.ShapeDtypeStruct((NW, RPT, DL), jnp.float32)),
        scratch_types=_IDX_SCRATCH + (
            pltpu.VMEM((B, DL), jnp.float32),
            pltpu.VMEM_SHARED((N, D), jnp.float32),
            pltpu.VMEM_SHARED((N, DL), jnp.float32),
        ) + _SEMS,
        **_SC_MESH,
    )(table, src, dst, zz, zd, ones_in)


def _sc_agg(table, src, dst, zz):
    return pl.kernel(
        _make_agg_body(False),
        out_type=jax.ShapeDtypeStruct((NW, RPT, D), jnp.float32),
        scratch_types=_IDX_SCRATCH + (
            pltpu.VMEM_SHARED((N, D), jnp.float32),
        ) + _SEMS,
        **_SC_MESH,
    )(table, src, dst, zz)


BLK = 1000
NBLK = N // BLK


def _layer_body(relu, x_ref, a0_ref, a1_ref, d0_ref, d1_ref, ws_ref, wn_ref,
                b_ref, o_ref):
    d = d0_ref[:, 0:1] + d1_ref[:, 0:1]
    rdeg = 1.0 / jnp.maximum(d, 1.0)
    neigh = (a0_ref[...] + a1_ref[...]) * rdeg
    h = (jnp.dot(x_ref[...], ws_ref[...], preferred_element_type=jnp.float32)
         + jnp.dot(neigh, wn_ref[...], preferred_element_type=jnp.float32)
         + b_ref[...])
    o_ref[...] = jnp.maximum(h, 0.0) if relu else h


def _tc_layer(x, agg, degp, ws_t, wn_t, b, relu):
    return pl.pallas_call(
        functools.partial(_layer_body, relu),
        grid=(NBLK,),
        in_specs=[
            pl.BlockSpec((BLK, D), lambda i: (i, 0)),
            pl.BlockSpec((BLK, D), lambda i: (i, 0)),
            pl.BlockSpec((BLK, D), lambda i: (i + NBLK, 0)),
            pl.BlockSpec((BLK, DL), lambda i: (i, 0)),
            pl.BlockSpec((BLK, DL), lambda i: (i + NBLK, 0)),
            pl.BlockSpec((D, D), lambda i: (0, 0)),
            pl.BlockSpec((D, D), lambda i: (0, 0)),
            pl.BlockSpec((1, D), lambda i: (0, 0)),
        ],
        out_specs=pl.BlockSpec((BLK, D), lambda i: (i, 0)),
        out_shape=jax.ShapeDtypeStruct((N, D), jnp.float32),
    )(x, agg, agg, degp, degp, ws_t, wn_t, b)


def kernel(x, edge_index, W_self1, W_neigh1, b1, W_self2, W_neigh2, b2):
    src = edge_index[0].astype(jnp.int32).reshape(NW * NB, B)
    dst = edge_index[1].astype(jnp.int32).reshape(NW * NB, B)
    zz = jnp.zeros((RPT, D), jnp.float32)
    zd = jnp.zeros((RPT, DL), jnp.float32)
    ones_in = jnp.ones((B, DL), jnp.float32)
    agg1, degp = _sc_agg_deg(x, src, dst, zz, zd, ones_in)
    agg1 = agg1.reshape(NC * N, D)
    degp = degp.reshape(NC * N, DL)
    h1 = _tc_layer(x, agg1, degp, W_self1.T, W_neigh1.T, b1[None, :], True)
    agg2 = _sc_agg(h1, src, dst, zz).reshape(NC * N, D)
    return _tc_layer(h1, agg2, degp, W_self2.T, W_neigh2.T, b2[None, :], False)


# speedup vs baseline: 1.0012x; 1.0012x over previous
import functools

import jax
import jax.numpy as jnp
from jax import lax
from jax.experimental import pallas as pl
from jax.experimental.pallas import tpu as pltpu
from jax.experimental.pallas import tpu_sc as plsc

N = 10000
D = 128
DL = 16
E = 320000
NC = 2
NS = 16
NW = NC * NS
EPW = E // NW
B = 125
NB = EPW // B
RPT = N // NS


def _make_agg_body(with_deg):
    def body(tab_hbm, sd_hbm, zz_hbm, *rest):
        if with_deg:
            (zd_hbm, ones_hbm, agg_hbm, deg_hbm,
             ib0, ib1, ib2, ib3, rows0, rows1, onesb, acc, degacc,
             semi0, semi1, semi2, semi3, semg0, semg1,
             sems0, sems1, semz) = rest
        else:
            (agg_hbm,
             ib0, ib1, ib2, ib3, rows0, rows1, acc,
             semi0, semi1, semi2, semi3, semg0, semg1,
             sems0, sems1, semz) = rest
        c = lax.axis_index("c")
        s = lax.axis_index("s")
        wid = s * NC + c
        pltpu.async_copy(zz_hbm, acc.at[pl.ds(s * RPT, RPT)], semz)
        if with_deg:
            pltpu.async_copy(zd_hbm, degacc.at[pl.ds(s * RPT, RPT)], semz)
            pltpu.async_copy(ones_hbm, onesb, semz)

        ibase = wid * NB

        def iload(i, ib, sem):
            pltpu.async_copy(sd_hbm.at[ibase + i], ib, sem)

        def iwait(ib, sem):
            pltpu.make_async_copy(sd_hbm.at[0], ib, sem).wait()

        def gather(ib, rows, sem):
            pltpu.async_copy(tab_hbm.at[ib.at[0]], rows, sem)

        def gwait(rows, sem):
            pltpu.make_async_copy(tab_hbm.at[ib0.at[0]], rows, sem).wait()

        def scat(ib, rows, sem):
            pltpu.async_copy(rows, acc.at[ib.at[1]], sem, add=True)
            if with_deg:
                pltpu.async_copy(onesb, degacc.at[ib.at[1]], sem, add=True)

        def swait(ib, rows, sem):
            pltpu.make_async_copy(rows, acc.at[ib.at[1]], sem).wait()
            if with_deg:
                pltpu.make_async_copy(onesb, degacc.at[ib.at[1]], sem).wait()

        iload(0, ib0, semi0)
        iload(1, ib1, semi1)
        iload(2, ib2, semi2)
        iload(3, ib3, semi3)
        iwait(ib0, semi0)
        gather(ib0, rows0, semg0)
        iwait(ib1, semi1)
        gather(ib1, rows1, semg1)
        pltpu.make_async_copy(zz_hbm, acc.at[pl.ds(s * RPT, RPT)],
                              semz).wait()
        if with_deg:
            pltpu.make_async_copy(zd_hbm, degacc.at[pl.ds(s * RPT, RPT)],
                                  semz).wait()
            pltpu.make_async_copy(ones_hbm, onesb, semz).wait()
        plsc.subcore_barrier()

        def quarter(i_next, ib_c, semi_c, ib_n, semi_n, rows, semg, sems):
            gwait(rows, semg)
            scat(ib_c, rows, sems)
            swait(ib_c, rows, sems)
            iload(i_next, ib_c, semi_c)
            iwait(ib_n, semi_n)
            gather(ib_n, rows, semg)

        def step(j, carry):
            i0 = 4 * j
            quarter(i0 + 4, ib0, semi0, ib2, semi2, rows0, semg0, sems0)
            quarter(i0 + 5, ib1, semi1, ib3, semi3, rows1, semg1, sems1)
            quarter(i0 + 6, ib2, semi2, ib0, semi0, rows0, semg0, sems0)
            quarter(i0 + 7, ib3, semi3, ib1, semi1, rows1, semg1, sems1)
            return carry

        lax.fori_loop(0, NB // 4 - 1, step, 0)
        gwait(rows0, semg0)
        scat(ib0, rows0, sems0)
        swait(ib0, rows0, sems0)
        iwait(ib2, semi2)
        gather(ib2, rows0, semg0)
        gwait(rows1, semg1)
        scat(ib1, rows1, sems1)
        swait(ib1, rows1, sems1)
        iwait(ib3, semi3)
        gather(ib3, rows1, semg1)
        gwait(rows0, semg0)
        scat(ib2, rows0, sems0)
        swait(ib2, rows0, sems0)
        gwait(rows1, semg1)
        scat(ib3, rows1, sems1)
        swait(ib3, rows1, sems1)
        plsc.subcore_barrier()
        pltpu.sync_copy(acc.at[pl.ds(s * RPT, RPT)], agg_hbm.at[c * NS + s])
        if with_deg:
            pltpu.sync_copy(degacc.at[pl.ds(s * RPT, RPT)],
                            deg_hbm.at[c * NS + s])

    return body


_SC_MESH = dict(
    mesh=plsc.VectorSubcoreMesh(core_axis_name="c", subcore_axis_name="s"),
    compiler_params=pltpu.CompilerParams(use_tc_tiling_on_sc=False),
)

_IDX_SCRATCH = (
    (pltpu.VMEM((2, B), jnp.int32),) * 4
    + (
        pltpu.VMEM((B, D), jnp.float32),
        pltpu.VMEM((B, D), jnp.float32),
    )
)

_SEMS = (pltpu.SemaphoreType.DMA,) * 9


def _sc_agg_deg(table, sd, zz, zd, ones_in):
    return pl.kernel(
        _make_agg_body(True),
        out_type=(jax.ShapeDtypeStruct((NW, RPT, D), jnp.float32),
                  jax.ShapeDtypeStruct((NW, RPT, DL), jnp.float32)),
        scratch_types=_IDX_SCRATCH + (
            pltpu.VMEM((B, DL), jnp.float32),
            pltpu.VMEM_SHARED((N, D), jnp.float32),
            pltpu.VMEM_SHARED((N, DL), jnp.float32),
        ) + _SEMS,
        **_SC_MESH,
    )(table, sd, zz, zd, ones_in)


def _sc_agg(table, sd, zz):
    return pl.kernel(
        _make_agg_body(False),
        out_type=jax.ShapeDtypeStruct((NW, RPT, D), jnp.float32),
        scratch_types=_IDX_SCRATCH + (
            pltpu.VMEM_SHARED((N, D), jnp.float32),
        ) + _SEMS,
        **_SC_MESH,
    )(table, sd, zz)


BLK = 1000
NBLK = N // BLK


def _layer_body(relu, x_ref, a0_ref, a1_ref, d0_ref, d1_ref, ws_ref, wn_ref,
                b_ref, o_ref):
    d = d0_ref[:, 0:1] + d1_ref[:, 0:1]
    rdeg = 1.0 / jnp.maximum(d, 1.0)
    neigh = (a0_ref[...] + a1_ref[...]) * rdeg
    h = (jnp.dot(x_ref[...], ws_ref[...], preferred_element_type=jnp.float32)
         + jnp.dot(neigh, wn_ref[...], preferred_element_type=jnp.float32)
         + b_ref[...])
    o_ref[...] = jnp.maximum(h, 0.0) if relu else h


def _tc_layer(x, agg, degp, ws_t, wn_t, b, relu):
    return pl.pallas_call(
        functools.partial(_layer_body, relu),
        grid=(NBLK,),
        in_specs=[
            pl.BlockSpec((BLK, D), lambda i: (i, 0)),
            pl.BlockSpec((BLK, D), lambda i: (i, 0)),
            pl.BlockSpec((BLK, D), lambda i: (i + NBLK, 0)),
            pl.BlockSpec((BLK, DL), lambda i: (i, 0)),
            pl.BlockSpec((BLK, DL), lambda i: (i + NBLK, 0)),
            pl.BlockSpec((D, D), lambda i: (0, 0)),
            pl.BlockSpec((D, D), lambda i: (0, 0)),
            pl.BlockSpec((1, D), lambda i: (0, 0)),
        ],
        out_specs=pl.BlockSpec((BLK, D), lambda i: (i, 0)),
        out_shape=jax.ShapeDtypeStruct((N, D), jnp.float32),
    )(x, agg, agg, degp, degp, ws_t, wn_t, b)


def kernel(x, edge_index, W_self1, W_neigh1, b1, W_self2, W_neigh2, b2):
    sd = jnp.stack([edge_index[0].astype(jnp.int32).reshape(NW * NB, B),
                    edge_index[1].astype(jnp.int32).reshape(NW * NB, B)],
                   axis=1)
    zz = jnp.zeros((RPT, D), jnp.float32)
    zd = jnp.zeros((RPT, DL), jnp.float32)
    ones_in = jnp.ones((B, DL), jnp.float32)
    agg1, degp = _sc_agg_deg(x, sd, zz, zd, ones_in)
    agg1 = agg1.reshape(NC * N, D)
    degp = degp.reshape(NC * N, DL)
    h1 = _tc_layer(x, agg1, degp, W_self1.T, W_neigh1.T, b1[None, :], True)
    agg2 = _sc_agg(h1, sd, zz).reshape(NC * N, D)
    return _tc_layer(h1, agg2, degp, W_self2.T, W_neigh2.T, b2[None, :], False)


# speedup vs baseline: 1.0613x; 1.0600x over previous
import functools

import jax
import jax.numpy as jnp
from jax import lax
from jax.experimental import pallas as pl
from jax.experimental.pallas import tpu as pltpu
from jax.experimental.pallas import tpu_sc as plsc

N = 10000
D = 128
DL = 16
E = 320000
NC = 2
NS = 16
NW = NC * NS
EPW = E // NW
B = 125
NB = EPW // B
RPT = 640
RPT_L = N - (NS - 1) * RPT


def _make_agg_body(with_deg):
    def body(tab_hbm, sd_hbm, zz_hbm, *rest):
        if with_deg:
            (zd_hbm, ones_hbm, agg_hbm, deg_hbm,
             ib0, ib1, ib2, ib3, rows0, rows1, onesb, acc, degacc,
             semi0, semi1, semi2, semi3, semg0, semg1,
             sems0, sems1, semz) = rest
        else:
            (agg_hbm,
             ib0, ib1, ib2, ib3, rows0, rows1, acc,
             semi0, semi1, semi2, semi3, semg0, semg1,
             sems0, sems1, semz) = rest
        c = lax.axis_index("c")
        s = lax.axis_index("s")
        wid = s * NC + c
        last = s == NS - 1

        def zero_start():
            @pl.when(~last)
            def _():
                pltpu.async_copy(zz_hbm, acc.at[pl.ds(s * RPT, RPT)], semz)
                if with_deg:
                    pltpu.async_copy(zd_hbm, degacc.at[pl.ds(s * RPT, RPT)],
                                     semz)

            @pl.when(last)
            def _():
                pltpu.async_copy(zz_hbm.at[pl.ds(0, RPT_L)],
                                 acc.at[pl.ds(s * RPT, RPT_L)], semz)
                if with_deg:
                    pltpu.async_copy(zd_hbm.at[pl.ds(0, RPT_L)],
                                     degacc.at[pl.ds(s * RPT, RPT_L)], semz)

        def zero_wait():
            @pl.when(~last)
            def _():
                pltpu.make_async_copy(zz_hbm, acc.at[pl.ds(s * RPT, RPT)],
                                      semz).wait()
                if with_deg:
                    pltpu.make_async_copy(
                        zd_hbm, degacc.at[pl.ds(s * RPT, RPT)], semz).wait()

            @pl.when(last)
            def _():
                pltpu.make_async_copy(zz_hbm.at[pl.ds(0, RPT_L)],
                                      acc.at[pl.ds(s * RPT, RPT_L)],
                                      semz).wait()
                if with_deg:
                    pltpu.make_async_copy(
                        zd_hbm.at[pl.ds(0, RPT_L)],
                        degacc.at[pl.ds(s * RPT, RPT_L)], semz).wait()

        zero_start()
        if with_deg:
            pltpu.async_copy(ones_hbm, onesb, semz)

        ibase = wid * NB

        def iload(i, ib, sem):
            pltpu.async_copy(sd_hbm.at[ibase + i], ib.at[0], sem)
            pltpu.async_copy(sd_hbm.at[NW * NB + ibase + i], ib.at[1], sem)

        def iwait(ib, sem):
            pltpu.make_async_copy(sd_hbm.at[0], ib.at[0], sem).wait()
            pltpu.make_async_copy(sd_hbm.at[0], ib.at[1], sem).wait()

        def gather(ib, rows, sem):
            pltpu.async_copy(tab_hbm.at[ib.at[0]], rows, sem)

        def gwait(rows, sem):
            pltpu.make_async_copy(tab_hbm.at[ib0.at[0]], rows, sem).wait()

        def scat(ib, rows, sem):
            pltpu.async_copy(rows, acc.at[ib.at[1]], sem, add=True)
            if with_deg:
                pltpu.async_copy(onesb, degacc.at[ib.at[1]], sem, add=True)

        def swait(ib, rows, sem):
            pltpu.make_async_copy(rows, acc.at[ib.at[1]], sem).wait()
            if with_deg:
                pltpu.make_async_copy(onesb, degacc.at[ib.at[1]], sem).wait()

        iload(0, ib0, semi0)
        iload(1, ib1, semi1)
        iload(2, ib2, semi2)
        iload(3, ib3, semi3)
        iwait(ib0, semi0)
        gather(ib0, rows0, semg0)
        iwait(ib1, semi1)
        gather(ib1, rows1, semg1)
        zero_wait()
        if with_deg:
            pltpu.make_async_copy(ones_hbm, onesb, semz).wait()
        plsc.subcore_barrier()

        def quarter(i_next, ib_c, semi_c, ib_n, semi_n, rows, semg, sems):
            gwait(rows, semg)
            scat(ib_c, rows, sems)
            swait(ib_c, rows, sems)
            iload(i_next, ib_c, semi_c)
            iwait(ib_n, semi_n)
            gather(ib_n, rows, semg)

        def step(j, carry):
            i0 = 4 * j
            quarter(i0 + 4, ib0, semi0, ib2, semi2, rows0, semg0, sems0)
            quarter(i0 + 5, ib1, semi1, ib3, semi3, rows1, semg1, sems1)
            quarter(i0 + 6, ib2, semi2, ib0, semi0, rows0, semg0, sems0)
            quarter(i0 + 7, ib3, semi3, ib1, semi1, rows1, semg1, sems1)
            return carry

        lax.fori_loop(0, NB // 4 - 1, step, 0)
        gwait(rows0, semg0)
        scat(ib0, rows0, sems0)
        swait(ib0, rows0, sems0)
        iwait(ib2, semi2)
        gather(ib2, rows0, semg0)
        gwait(rows1, semg1)
        scat(ib1, rows1, sems1)
        swait(ib1, rows1, sems1)
        iwait(ib3, semi3)
        gather(ib3, rows1, semg1)
        gwait(rows0, semg0)
        scat(ib2, rows0, sems0)
        swait(ib2, rows0, sems0)
        gwait(rows1, semg1)
        scat(ib3, rows1, sems1)
        swait(ib3, rows1, sems1)
        plsc.subcore_barrier()

        @pl.when(~last)
        def _():
            pltpu.sync_copy(acc.at[pl.ds(s * RPT, RPT)],
                            agg_hbm.at[pl.ds(c * N + s * RPT, RPT)])
            if with_deg:
                pltpu.sync_copy(degacc.at[pl.ds(s * RPT, RPT)],
                                deg_hbm.at[pl.ds(c * N + s * RPT, RPT)])

        @pl.when(last)
        def _():
            pltpu.sync_copy(acc.at[pl.ds(s * RPT, RPT_L)],
                            agg_hbm.at[pl.ds(c * N + s * RPT, RPT_L)])
            if with_deg:
                pltpu.sync_copy(degacc.at[pl.ds(s * RPT, RPT_L)],
                                deg_hbm.at[pl.ds(c * N + s * RPT, RPT_L)])

    return body


_SC_MESH = dict(
    mesh=plsc.VectorSubcoreMesh(core_axis_name="c", subcore_axis_name="s"),
    compiler_params=pltpu.CompilerParams(use_tc_tiling_on_sc=False),
)

_IDX_SCRATCH = (
    (pltpu.VMEM((2, B), jnp.int32),) * 4
    + (
        pltpu.VMEM((B, D), jnp.float32),
        pltpu.VMEM((B, D), jnp.float32),
    )
)

_SEMS = (pltpu.SemaphoreType.DMA,) * 9


def _sc_agg_deg(table, sd, zz, zd, ones_in):
    return pl.kernel(
        _make_agg_body(True),
        out_type=(jax.ShapeDtypeStruct((NC * N, D), jnp.float32),
                  jax.ShapeDtypeStruct((NC * N, DL), jnp.float32)),
        scratch_types=_IDX_SCRATCH + (
            pltpu.VMEM((B, DL), jnp.float32),
            pltpu.VMEM_SHARED((N, D), jnp.float32),
            pltpu.VMEM_SHARED((N, DL), jnp.float32),
        ) + _SEMS,
        **_SC_MESH,
    )(table, sd, zz, zd, ones_in)


def _sc_agg(table, sd, zz):
    return pl.kernel(
        _make_agg_body(False),
        out_type=jax.ShapeDtypeStruct((NC * N, D), jnp.float32),
        scratch_types=_IDX_SCRATCH + (
            pltpu.VMEM_SHARED((N, D), jnp.float32),
        ) + _SEMS,
        **_SC_MESH,
    )(table, sd, zz)


BLK = 1000
NBLK = N // BLK


def _layer_body(relu, x_ref, a0_ref, a1_ref, d0_ref, d1_ref, ws_ref, wn_ref,
                b_ref, o_ref):
    d = d0_ref[:, 0:1] + d1_ref[:, 0:1]
    rdeg = 1.0 / jnp.maximum(d, 1.0)
    neigh = (a0_ref[...] + a1_ref[...]) * rdeg
    h = (jnp.dot(x_ref[...], ws_ref[...], preferred_element_type=jnp.float32)
         + jnp.dot(neigh, wn_ref[...], preferred_element_type=jnp.float32)
         + b_ref[...])
    o_ref[...] = jnp.maximum(h, 0.0) if relu else h


def _tc_layer(x, agg, degp, ws_t, wn_t, b, relu):
    return pl.pallas_call(
        functools.partial(_layer_body, relu),
        grid=(NBLK,),
        in_specs=[
            pl.BlockSpec((BLK, D), lambda i: (i, 0)),
            pl.BlockSpec((BLK, D), lambda i: (i, 0)),
            pl.BlockSpec((BLK, D), lambda i: (i + NBLK, 0)),
            pl.BlockSpec((BLK, DL), lambda i: (i, 0)),
            pl.BlockSpec((BLK, DL), lambda i: (i + NBLK, 0)),
            pl.BlockSpec((D, D), lambda i: (0, 0)),
            pl.BlockSpec((D, D), lambda i: (0, 0)),
            pl.BlockSpec((1, D), lambda i: (0, 0)),
        ],
        out_specs=pl.BlockSpec((BLK, D), lambda i: (i, 0)),
        out_shape=jax.ShapeDtypeStruct((N, D), jnp.float32),
    )(x, agg, agg, degp, degp, ws_t, wn_t, b)


def kernel(x, edge_index, W_self1, W_neigh1, b1, W_self2, W_neigh2, b2):
    sd = edge_index.astype(jnp.int32).reshape(2 * NW * NB, B)
    zz = jnp.zeros((RPT, D), jnp.float32)
    zd = jnp.zeros((RPT, DL), jnp.float32)
    ones_in = jnp.ones((B, DL), jnp.float32)
    agg1, degp = _sc_agg_deg(x, sd, zz, zd, ones_in)
    h1 = _tc_layer(x, agg1, degp, W_self1.T, W_neigh1.T, b1[None, :], True)
    agg2 = _sc_agg(h1, sd, zz)
    return _tc_layer(h1, agg2, degp, W_self2.T, W_neigh2.T, b2[None, :], False)
